# baseline (speedup 1.0000x reference)
import functools

import jax
import jax.numpy as jnp
from jax import lax
from jax.experimental import pallas as pl
from jax.experimental.pallas import tpu as pltpu
from jax.experimental.pallas import tpu_sc as plsc

DIM = 32
NUM_WORKERS = 32
ROWS_PER_BLK = 512
BLK_SHIFT = 9
CHUNKROWS = 128


def _worker_id():
    return lax.axis_index("s") * 2 + lax.axis_index("c")


def _scan_extract_kernel(batch, n_rows):
    nblk = n_rows // ROWS_PER_BLK
    tail_base = nblk * ROWS_PER_BLK
    tail_len = n_rows - tail_base
    base_per_w = nblk // NUM_WORKERS
    extra = nblk - base_per_w * NUM_WORKERS
    mesh = plsc.VectorSubcoreMesh(core_axis_name="c", subcore_axis_name="s")

    @functools.partial(
        pl.kernel,
        mesh=mesh,
        out_type=(
            jax.ShapeDtypeStruct((batch, 128), jnp.float32),
            jax.ShapeDtypeStruct((batch, 128), jnp.float32),
        ),
        scratch_types=[
            pltpu.VMEM((batch,), jnp.int32),
            pltpu.VMEM((batch,), jnp.int32),
            pltpu.VMEM((batch,), jnp.int32),
            pltpu.VMEM((2, DIM, ROWS_PER_BLK), jnp.float32),
            pltpu.VMEM((CHUNKROWS, 128), jnp.float32),
            pltpu.VMEM((2, CHUNKROWS), jnp.int32),
            pltpu.SemaphoreType.DMA,
            pltpu.SemaphoreType.DMA,
        ],
    )
    def k(user_hbm, item_hbm, UT_hbm, IT_hbm, uscr_hbm, iscr_hbm,
          idxstage, rbuf, bbuf, dbuf, rows_v, brec, sem_blk, sem_sct):
        w = _worker_id()
        bstart = w * base_per_w + jnp.minimum(w, extra)
        nfull = base_per_w + (w < extra).astype(jnp.int32)
        has_tail = w == NUM_WORKERS - 1
        bhi = bstart + nfull + has_tail.astype(jnp.int32)

        lane = lax.iota(jnp.int32, 16)
        ones16 = jnp.ones((16,), jnp.int32)
        neg16 = jnp.full((16,), -1, jnp.int32)

        def reset_brec():
            for t in range(CHUNKROWS // 16):
                brec[0, pl.ds(t * 16, 16)] = neg16

        def make_flush(scr_hbm):
            def flush(fill):
                pltpu.async_copy(
                    rows_v,
                    scr_hbm.at[plsc.Indices(brec.at[0], ignored_value=-1)],
                    sem_sct,
                ).wait()
                reset_brec()
                return fill * 0
            return flush

        for idx_hbm, T_hbm, scr_hbm in (
            (user_hbm, UT_hbm, uscr_hbm),
            (item_hbm, IT_hbm, iscr_hbm),
        ):
            flush = make_flush(scr_hbm)

            pltpu.sync_copy(idx_hbm, idxstage)

            def filt(t, off):
                rv = idxstage[pl.ds(t * 16, 16)]
                blk = lax.shift_right_logical(rv, BLK_SHIFT)
                mine = (blk >= bstart) & (blk < bhi)
                anyv = jnp.where(mine, ones16, 0)
                for sh in (8, 4, 2, 1):
                    anyv = anyv | anyv.at[lane ^ sh].get(
                        mode="promise_in_bounds")

                def compact(off):
                    x = jnp.where(mine, ones16, 0)
                    for sh in (1, 2, 4, 8):
                        g = x.at[jnp.maximum(lane - sh, 0)].get(
                            mode="promise_in_bounds")
                        x = x + jnp.where(lane >= sh, g, 0)
                    pos = off + x - jnp.where(mine, ones16, 0)
                    bv = t * 16 + lane
                    mine32 = jnp.where(mine, ones16, 0)
                    for l in range(16):
                        def wr(off, l=l):
                            p = pos[l]
                            base = pl.multiple_of(p - (p & 15), 16)
                            slot = lane == (p & 15)
                            cr = rbuf[pl.ds(base, 16)]
                            rbuf[pl.ds(base, 16)] = jnp.where(slot, rv[l], cr)
                            cb = bbuf[pl.ds(base, 16)]
                            bbuf[pl.ds(base, 16)] = jnp.where(slot, bv[l], cb)
                            return off
                        off = lax.cond(mine32[l] != 0, wr, lambda o: o, off)
                    return off + x[15]

                return lax.cond(anyv[0] != 0, compact, lambda o: o, off)

            m = lax.fori_loop(0, batch // 16, filt, jnp.int32(0))
            nch = lax.div(m + 15, jnp.int32(16))

            reset_brec()

            def scan_block(kg, kb, fill):
                def chunk(t, fill):
                    rv = rbuf[pl.ds(t * 16, 16)]
                    valid = lane < (m - t * 16)
                    mask = (lax.shift_right_logical(rv, BLK_SHIFT) == kg)
                    mask = mask & valid
                    cnt = jnp.where(mask, ones16, 0)
                    for sh in (8, 4, 2, 1):
                        cnt = cnt | cnt.at[lane ^ sh].get(
                            mode="promise_in_bounds")

                    def any_match(fill):
                        mask32 = jnp.where(mask, ones16, 0)
                        bv = bbuf[pl.ds(t * 16, 16)]
                        rr = rv & (ROWS_PER_BLK - 1)
                        for l in range(16):
                            def extract(fill, l=l):
                                base16 = pl.multiple_of(
                                    rr[l] - (rr[l] & 15), 16)
                                pick = jnp.full((16,), rr[l] & 15)
                                row0 = jnp.zeros((16,), jnp.float32)
                                row1 = jnp.zeros((16,), jnp.float32)
                                for c in range(DIM):
                                    run = dbuf[kb, c, pl.ds(base16, 16)]
                                    val = run.at[pick].get(
                                        mode="promise_in_bounds")
                                    if c < 16:
                                        row0 = jnp.where(lane == c, val, row0)
                                    else:
                                        row1 = jnp.where(
                                            lane == c - 16, val, row1)
                                rows_v[fill, pl.ds(0, 16)] = row0
                                rows_v[fill, pl.ds(16, 16)] = row1
                                bbase = pl.multiple_of(
                                    fill - (fill & 15), 16)
                                cbr = brec[0, pl.ds(bbase, 16)]
                                brec[0, pl.ds(bbase, 16)] = jnp.where(
                                    lane == (fill & 15), bv[l], cbr)
                                return fill + 1

                            fill = lax.cond(
                                mask32[l] != 0, extract,
                                lambda fill: fill, fill)
                        return fill

                    fill = lax.cond(cnt[0] != 0, any_match,
                                    lambda fill: fill, fill)
                    fill = lax.cond(fill >= CHUNKROWS - 16,
                                    flush, lambda f: f, fill)
                    return fill

                return lax.fori_loop(0, nch, chunk, fill)

            first_off = pl.multiple_of(bstart * ROWS_PER_BLK, ROWS_PER_BLK)
            pltpu.async_copy(
                T_hbm.at[:, pl.ds(first_off, ROWS_PER_BLK)],
                dbuf.at[0], sem_blk)

            def blk_body(j, fill):
                kb = j & 1
                pltpu.make_async_copy(
                    T_hbm.at[:, pl.ds(0, ROWS_PER_BLK)],
                    dbuf.at[kb], sem_blk).wait()

                @pl.when(j + 1 < nfull)
                def _():
                    noff = pl.multiple_of(
                        (bstart + j + 1) * ROWS_PER_BLK, ROWS_PER_BLK)
                    pltpu.async_copy(
                        T_hbm.at[:, pl.ds(noff, ROWS_PER_BLK)],
                        dbuf.at[(j + 1) & 1], sem_blk)

                return scan_block(bstart + j, kb, fill)

            fill = lax.fori_loop(0, nfull, blk_body, jnp.int32(0))

            @pl.when(has_tail)
            def _():
                toff = pl.multiple_of(tail_base + w * 0, 128)
                pltpu.sync_copy(
                    T_hbm.at[:, pl.ds(toff, 128)],
                    dbuf.at[0].at[:, pl.ds(0, 128)])

            fill = lax.cond(
                has_tail,
                lambda f: scan_block(jnp.int32(nblk), jnp.int32(0), f),
                lambda f: f, fill)

            lax.cond(fill > 0, flush, lambda f: f, fill)

    return k


def _dot_kernel(batch):
    b_per_w = batch // NUM_WORKERS
    nchunk = b_per_w // 128
    mesh = plsc.VectorSubcoreMesh(core_axis_name="c", subcore_axis_name="s")

    @functools.partial(
        pl.kernel,
        mesh=mesh,
        out_type=jax.ShapeDtypeStruct((batch,), jnp.float32),
        scratch_types=[
            pltpu.VMEM((2, 128, 128), jnp.float32),
            pltpu.VMEM((2, 128, 128), jnp.float32),
            pltpu.VMEM((b_per_w,), jnp.float32),
            pltpu.SemaphoreType.DMA,
        ],
    )
    def k(uscr_hbm, iscr_hbm, out_hbm, urows_v, irows_v, out_v, sem):
        w = _worker_id()
        base = w * b_per_w

        def start(j):
            sl = pl.ds(base + j * 128, 128)
            cu = pltpu.async_copy(uscr_hbm.at[sl, :], urows_v.at[j % 2], sem)
            ci = pltpu.async_copy(iscr_hbm.at[sl, :], irows_v.at[j % 2], sem)
            return cu, ci

        lane = lax.iota(jnp.int32, 16)
        perms = [lane ^ sh for sh in (8, 4, 2, 1)]

        inflight = start(0)
        for j in range(nchunk):
            for c in inflight:
                c.wait()
            if j + 1 < nchunk:
                inflight = start(j + 1)
            cu = urows_v.at[j % 2]
            ci = irows_v.at[j % 2]

            def grp_body(g, _, cu=cu, ci=ci, j=j):
                acc = jnp.zeros((16,), jnp.float32)
                for r in range(16):
                    row = g * 16 + r
                    u0 = cu[row, pl.ds(0, 16)]
                    u1 = cu[row, pl.ds(16, 16)]
                    i0 = ci[row, pl.ds(0, 16)]
                    i1 = ci[row, pl.ds(16, 16)]
                    p = u0 * i0 + u1 * i1
                    for pm in perms:
                        p = p + p.at[pm].get(mode="promise_in_bounds")
                    acc = jnp.where(lane == r, p, acc)
                out_v[pl.ds(j * 128 + g * 16, 16)] = acc
                return 0

            lax.fori_loop(0, 8, grp_body, 0)

        pltpu.sync_copy(out_v, out_hbm.at[pl.ds(base, b_per_w)])

    return k


@functools.partial(jax.jit, static_argnames=("batch", "n_u", "n_i"))
def _mf(user, item, U, I, batch, n_u, n_i):
    UT = jnp.transpose(U)
    IT = jnp.transpose(I)
    uscr, iscr = _scan_extract_kernel(batch, n_u)(user, item, UT, IT)
    return _dot_kernel(batch)(uscr, iscr)


def kernel(user, item, U, I):
    return _mf(user, item, U, I, user.shape[0], U.shape[0], I.shape[0])


# speedup vs baseline: 1.0102x; 1.0102x over previous
import functools

import jax
import jax.numpy as jnp
from jax import lax
from jax.experimental import pallas as pl
from jax.experimental.pallas import tpu as pltpu
from jax.experimental.pallas import tpu_sc as plsc

DIM = 32
NUM_WORKERS = 32
ROWS_PER_BLK = 512
BLK_SHIFT = 9
CHUNKROWS = 128


def _worker_id():
    return lax.axis_index("s") * 2 + lax.axis_index("c")


def _scan_extract_kernel(batch, n_rows):
    nblk = n_rows // ROWS_PER_BLK
    tail_base = nblk * ROWS_PER_BLK
    tail_len = n_rows - tail_base
    base_per_w = nblk // NUM_WORKERS
    extra = nblk - base_per_w * NUM_WORKERS
    mesh = plsc.VectorSubcoreMesh(core_axis_name="c", subcore_axis_name="s")

    @functools.partial(
        pl.kernel,
        mesh=mesh,
        out_type=(
            jax.ShapeDtypeStruct((batch, 128), jnp.float32),
            jax.ShapeDtypeStruct((batch, 128), jnp.float32),
        ),
        scratch_types=[
            pltpu.VMEM((batch,), jnp.int32),
            pltpu.VMEM((batch,), jnp.int32),
            pltpu.VMEM((batch,), jnp.int32),
            pltpu.VMEM((2, DIM, ROWS_PER_BLK), jnp.float32),
            pltpu.VMEM((CHUNKROWS, 128), jnp.float32),
            pltpu.VMEM((2, CHUNKROWS), jnp.int32),
            pltpu.SemaphoreType.DMA,
            pltpu.SemaphoreType.DMA,
        ],
    )
    def k(user_hbm, item_hbm, UT_hbm, IT_hbm, uscr_hbm, iscr_hbm,
          idxstage, rbuf, bbuf, dbuf, rows_v, brec, sem_blk, sem_sct):
        w = _worker_id()
        bstart = w * base_per_w + jnp.minimum(w, extra)
        nfull = base_per_w + (w < extra).astype(jnp.int32)
        has_tail = w == NUM_WORKERS - 1
        bhi = bstart + nfull + has_tail.astype(jnp.int32)

        lane = lax.iota(jnp.int32, 16)
        ones16 = jnp.ones((16,), jnp.int32)
        neg16 = jnp.full((16,), -1, jnp.int32)

        def reset_brec():
            for t in range(CHUNKROWS // 16):
                brec[0, pl.ds(t * 16, 16)] = neg16

        def make_flush(scr_hbm):
            def flush(fill):
                pltpu.async_copy(
                    rows_v,
                    scr_hbm.at[plsc.Indices(brec.at[0], ignored_value=-1)],
                    sem_sct,
                ).wait()
                reset_brec()
                return fill * 0
            return flush

        for idx_hbm, T_hbm, scr_hbm in (
            (user_hbm, UT_hbm, uscr_hbm),
            (item_hbm, IT_hbm, iscr_hbm),
        ):
            flush = make_flush(scr_hbm)

            pltpu.sync_copy(idx_hbm, idxstage)

            def filt(t, off):
                rv = idxstage[pl.ds(t * 16, 16)]
                blk = lax.shift_right_logical(rv, BLK_SHIFT)
                mine = (blk >= bstart) & (blk < bhi)
                anyv = jnp.where(mine, ones16, 0)
                for sh in (8, 4, 2, 1):
                    anyv = anyv | anyv.at[lane ^ sh].get(
                        mode="promise_in_bounds")

                def compact(off):
                    x = jnp.where(mine, ones16, 0)
                    for sh in (1, 2, 4, 8):
                        g = x.at[jnp.maximum(lane - sh, 0)].get(
                            mode="promise_in_bounds")
                        x = x + jnp.where(lane >= sh, g, 0)
                    pos = off + x - jnp.where(mine, ones16, 0)
                    bv = t * 16 + lane
                    mine32 = jnp.where(mine, ones16, 0)
                    for l in range(16):
                        def wr(off, l=l):
                            p = pos[l]
                            base = pl.multiple_of(p - (p & 15), 16)
                            slot = lane == (p & 15)
                            cr = rbuf[pl.ds(base, 16)]
                            rbuf[pl.ds(base, 16)] = jnp.where(slot, rv[l], cr)
                            cb = bbuf[pl.ds(base, 16)]
                            bbuf[pl.ds(base, 16)] = jnp.where(slot, bv[l], cb)
                            return off
                        off = lax.cond(mine32[l] != 0, wr, lambda o: o, off)
                    return off + x[15]

                return lax.cond(anyv[0] != 0, compact, lambda o: o, off)

            m = lax.fori_loop(0, batch // 16, filt, jnp.int32(0))
            nch = lax.div(m + 15, jnp.int32(16))

            reset_brec()

            def scan_block(kg, kb, fill):
                def chunk(t, fill):
                    rv = rbuf[pl.ds(t * 16, 16)]
                    valid = lane < (m - t * 16)
                    mask = (lax.shift_right_logical(rv, BLK_SHIFT) == kg)
                    mask = mask & valid
                    cnt = jnp.where(mask, ones16, 0)
                    for sh in (8, 4, 2, 1):
                        cnt = cnt | cnt.at[lane ^ sh].get(
                            mode="promise_in_bounds")

                    def any_match(fill):
                        mask32 = jnp.where(mask, ones16, 0)
                        bv = bbuf[pl.ds(t * 16, 16)]
                        rr = rv & (ROWS_PER_BLK - 1)
                        for l in range(16):
                            def extract(fill, l=l):
                                base16 = pl.multiple_of(
                                    rr[l] - (rr[l] & 15), 16)
                                pick = jnp.full((16,), rr[l] & 15)
                                row0 = jnp.zeros((16,), jnp.float32)
                                row1 = jnp.zeros((16,), jnp.float32)
                                for c in range(DIM):
                                    run = dbuf[kb, c, pl.ds(base16, 16)]
                                    val = run.at[pick].get(
                                        mode="promise_in_bounds")
                                    if c < 16:
                                        row0 = jnp.where(lane == c, val, row0)
                                    else:
                                        row1 = jnp.where(
                                            lane == c - 16, val, row1)
                                rows_v[fill, pl.ds(0, 16)] = row0
                                rows_v[fill, pl.ds(16, 16)] = row1
                                bbase = pl.multiple_of(
                                    fill - (fill & 15), 16)
                                cbr = brec[0, pl.ds(bbase, 16)]
                                brec[0, pl.ds(bbase, 16)] = jnp.where(
                                    lane == (fill & 15), bv[l], cbr)
                                return fill + 1

                            fill = lax.cond(
                                mask32[l] != 0, extract,
                                lambda fill: fill, fill)
                        return fill

                    fill = lax.cond(cnt[0] != 0, any_match,
                                    lambda fill: fill, fill)
                    fill = lax.cond(fill >= CHUNKROWS - 16,
                                    flush, lambda f: f, fill)
                    return fill

                return lax.fori_loop(0, nch, chunk, fill)

            first_off = pl.multiple_of(bstart * ROWS_PER_BLK, ROWS_PER_BLK)
            pltpu.async_copy(
                T_hbm.at[:, pl.ds(first_off, ROWS_PER_BLK)],
                dbuf.at[0], sem_blk)

            nloop = nfull + has_tail.astype(jnp.int32)

            def blk_body(j, fill):
                kb = j & 1
                pltpu.make_async_copy(
                    T_hbm.at[:, pl.ds(0, ROWS_PER_BLK)],
                    dbuf.at[kb], sem_blk).wait()

                @pl.when(j + 1 < nloop)
                def _():
                    noff = pl.multiple_of(
                        (bstart + j + 1) * ROWS_PER_BLK, ROWS_PER_BLK)
                    pltpu.async_copy(
                        T_hbm.at[:, pl.ds(noff, ROWS_PER_BLK)],
                        dbuf.at[(j + 1) & 1], sem_blk)

                return scan_block(bstart + j, kb, fill)

            fill = lax.fori_loop(0, nloop, blk_body, jnp.int32(0))

            lax.cond(fill > 0, flush, lambda f: f, fill)

    return k


def _dot_kernel(batch):
    b_per_w = batch // NUM_WORKERS
    nchunk = b_per_w // 128
    mesh = plsc.VectorSubcoreMesh(core_axis_name="c", subcore_axis_name="s")

    @functools.partial(
        pl.kernel,
        mesh=mesh,
        out_type=jax.ShapeDtypeStruct((batch,), jnp.float32),
        scratch_types=[
            pltpu.VMEM((2, 128, 128), jnp.float32),
            pltpu.VMEM((2, 128, 128), jnp.float32),
            pltpu.VMEM((b_per_w,), jnp.float32),
            pltpu.SemaphoreType.DMA,
        ],
    )
    def k(uscr_hbm, iscr_hbm, out_hbm, urows_v, irows_v, out_v, sem):
        w = _worker_id()
        base = w * b_per_w

        def start(j):
            sl = pl.ds(base + j * 128, 128)
            cu = pltpu.async_copy(uscr_hbm.at[sl, :], urows_v.at[j % 2], sem)
            ci = pltpu.async_copy(iscr_hbm.at[sl, :], irows_v.at[j % 2], sem)
            return cu, ci

        lane = lax.iota(jnp.int32, 16)
        perms = [lane ^ sh for sh in (8, 4, 2, 1)]

        inflight = start(0)
        for j in range(nchunk):
            for c in inflight:
                c.wait()
            if j + 1 < nchunk:
                inflight = start(j + 1)
            cu = urows_v.at[j % 2]
            ci = irows_v.at[j % 2]

            def grp_body(g, _, cu=cu, ci=ci, j=j):
                acc = jnp.zeros((16,), jnp.float32)
                for r in range(16):
                    row = g * 16 + r
                    u0 = cu[row, pl.ds(0, 16)]
                    u1 = cu[row, pl.ds(16, 16)]
                    i0 = ci[row, pl.ds(0, 16)]
                    i1 = ci[row, pl.ds(16, 16)]
                    p = u0 * i0 + u1 * i1
                    for pm in perms:
                        p = p + p.at[pm].get(mode="promise_in_bounds")
                    acc = jnp.where(lane == r, p, acc)
                out_v[pl.ds(j * 128 + g * 16, 16)] = acc
                return 0

            lax.fori_loop(0, 8, grp_body, 0)

        pltpu.sync_copy(out_v, out_hbm.at[pl.ds(base, b_per_w)])

    return k


@functools.partial(jax.jit, static_argnames=("batch", "n_u", "n_i"))
def _mf(user, item, U, I, batch, n_u, n_i):
    UT = jnp.transpose(U)
    IT = jnp.transpose(I)
    uscr, iscr = _scan_extract_kernel(batch, n_u)(user, item, UT, IT)
    return _dot_kernel(batch)(uscr, iscr)


def kernel(user, item, U, I):
    return _mf(user, item, U, I, user.shape[0], U.shape[0], I.shape[0])


# speedup vs baseline: 1.0218x; 1.0115x over previous
import functools

import jax
import jax.numpy as jnp
from jax import lax
from jax.experimental import pallas as pl
from jax.experimental.pallas import tpu as pltpu
from jax.experimental.pallas import tpu_sc as plsc

DIM = 32
NUM_WORKERS = 32
ROWS_PER_BLK = 512
BLK_SHIFT = 9
CHUNKROWS = 64
NBUF = 4


def _worker_id():
    return lax.axis_index("s") * 2 + lax.axis_index("c")


def _scan_extract_kernel(batch, n_rows):
    nblk = n_rows // ROWS_PER_BLK
    tail_base = nblk * ROWS_PER_BLK
    tail_len = n_rows - tail_base
    base_per_w = nblk // NUM_WORKERS
    extra = nblk - base_per_w * NUM_WORKERS
    mesh = plsc.VectorSubcoreMesh(core_axis_name="c", subcore_axis_name="s")

    @functools.partial(
        pl.kernel,
        mesh=mesh,
        out_type=(
            jax.ShapeDtypeStruct((batch, 128), jnp.float32),
            jax.ShapeDtypeStruct((batch, 128), jnp.float32),
        ),
        scratch_types=[
            pltpu.VMEM((batch,), jnp.int32),
            pltpu.VMEM((batch,), jnp.int32),
            pltpu.VMEM((batch,), jnp.int32),
            pltpu.VMEM((NBUF, DIM, ROWS_PER_BLK), jnp.float32),
            pltpu.VMEM((CHUNKROWS, 128), jnp.float32),
            pltpu.VMEM((2, CHUNKROWS), jnp.int32),
            pltpu.SemaphoreType.DMA,
            pltpu.SemaphoreType.DMA,
        ],
    )
    def k(user_hbm, item_hbm, UT_hbm, IT_hbm, uscr_hbm, iscr_hbm,
          idxstage, rbuf, bbuf, dbuf, rows_v, brec, sem_blk, sem_sct):
        w = _worker_id()
        bstart = w * base_per_w + jnp.minimum(w, extra)
        nfull = base_per_w + (w < extra).astype(jnp.int32)
        has_tail = w == NUM_WORKERS - 1
        bhi = bstart + nfull + has_tail.astype(jnp.int32)

        lane = lax.iota(jnp.int32, 16)
        ones16 = jnp.ones((16,), jnp.int32)
        neg16 = jnp.full((16,), -1, jnp.int32)

        def reset_brec():
            for t in range(CHUNKROWS // 16):
                brec[0, pl.ds(t * 16, 16)] = neg16

        def make_flush(scr_hbm):
            def flush(fill):
                pltpu.async_copy(
                    rows_v,
                    scr_hbm.at[plsc.Indices(brec.at[0], ignored_value=-1)],
                    sem_sct,
                ).wait()
                reset_brec()
                return fill * 0
            return flush

        for idx_hbm, T_hbm, scr_hbm in (
            (user_hbm, UT_hbm, uscr_hbm),
            (item_hbm, IT_hbm, iscr_hbm),
        ):
            flush = make_flush(scr_hbm)

            pltpu.sync_copy(idx_hbm, idxstage)

            def filt(t, off):
                rv = idxstage[pl.ds(t * 16, 16)]
                blk = lax.shift_right_logical(rv, BLK_SHIFT)
                mine = (blk >= bstart) & (blk < bhi)
                anyv = jnp.where(mine, ones16, 0)
                for sh in (8, 4, 2, 1):
                    anyv = anyv | anyv.at[lane ^ sh].get(
                        mode="promise_in_bounds")

                def compact(off):
                    x = jnp.where(mine, ones16, 0)
                    for sh in (1, 2, 4, 8):
                        g = x.at[jnp.maximum(lane - sh, 0)].get(
                            mode="promise_in_bounds")
                        x = x + jnp.where(lane >= sh, g, 0)
                    pos = off + x - jnp.where(mine, ones16, 0)
                    bv = t * 16 + lane
                    mine32 = jnp.where(mine, ones16, 0)
                    for l in range(16):
                        def wr(off, l=l):
                            p = pos[l]
                            base = pl.multiple_of(p - (p & 15), 16)
                            slot = lane == (p & 15)
                            cr = rbuf[pl.ds(base, 16)]
                            rbuf[pl.ds(base, 16)] = jnp.where(slot, rv[l], cr)
                            cb = bbuf[pl.ds(base, 16)]
                            bbuf[pl.ds(base, 16)] = jnp.where(slot, bv[l], cb)
                            return off
                        off = lax.cond(mine32[l] != 0, wr, lambda o: o, off)
                    return off + x[15]

                return lax.cond(anyv[0] != 0, compact, lambda o: o, off)

            m = lax.fori_loop(0, batch // 16, filt, jnp.int32(0))
            nch = lax.div(m + 15, jnp.int32(16))

            reset_brec()

            def scan_block(kg, kb, fill):
                def chunk(t, fill):
                    rv = rbuf[pl.ds(t * 16, 16)]
                    valid = lane < (m - t * 16)
                    mask = (lax.shift_right_logical(rv, BLK_SHIFT) == kg)
                    mask = mask & valid
                    cnt = jnp.where(mask, ones16, 0)
                    for sh in (8, 4, 2, 1):
                        cnt = cnt | cnt.at[lane ^ sh].get(
                            mode="promise_in_bounds")

                    def any_match(fill):
                        mask32 = jnp.where(mask, ones16, 0)
                        bv = bbuf[pl.ds(t * 16, 16)]
                        rr = rv & (ROWS_PER_BLK - 1)
                        for l in range(16):
                            def extract(fill, l=l):
                                base16 = pl.multiple_of(
                                    rr[l] - (rr[l] & 15), 16)
                                pick = jnp.full((16,), rr[l] & 15)
                                row0 = jnp.zeros((16,), jnp.float32)
                                row1 = jnp.zeros((16,), jnp.float32)
                                for c in range(DIM):
                                    run = dbuf[kb, c, pl.ds(base16, 16)]
                                    val = run.at[pick].get(
                                        mode="promise_in_bounds")
                                    if c < 16:
                                        row0 = jnp.where(lane == c, val, row0)
                                    else:
                                        row1 = jnp.where(
                                            lane == c - 16, val, row1)
                                rows_v[fill, pl.ds(0, 16)] = row0
                                rows_v[fill, pl.ds(16, 16)] = row1
                                bbase = pl.multiple_of(
                                    fill - (fill & 15), 16)
                                cbr = brec[0, pl.ds(bbase, 16)]
                                brec[0, pl.ds(bbase, 16)] = jnp.where(
                                    lane == (fill & 15), bv[l], cbr)
                                return fill + 1

                            fill = lax.cond(
                                mask32[l] != 0, extract,
                                lambda fill: fill, fill)
                        return fill

                    fill = lax.cond(cnt[0] != 0, any_match,
                                    lambda fill: fill, fill)
                    fill = lax.cond(fill >= CHUNKROWS - 16,
                                    flush, lambda f: f, fill)
                    return fill

                return lax.fori_loop(0, nch, chunk, fill)


            nloop = nfull + has_tail.astype(jnp.int32)

            for jj in range(NBUF - 1):
                poff = pl.multiple_of(
                    (bstart + jj) * ROWS_PER_BLK, ROWS_PER_BLK)
                pltpu.async_copy(
                    T_hbm.at[:, pl.ds(poff, ROWS_PER_BLK)],
                    dbuf.at[jj], sem_blk)

            def blk_body(j, fill):
                kb = j & (NBUF - 1)
                pltpu.make_async_copy(
                    T_hbm.at[:, pl.ds(0, ROWS_PER_BLK)],
                    dbuf.at[kb], sem_blk).wait()

                @pl.when(j + NBUF - 1 < nloop)
                def _():
                    noff = pl.multiple_of(
                        (bstart + j + NBUF - 1) * ROWS_PER_BLK, ROWS_PER_BLK)
                    pltpu.async_copy(
                        T_hbm.at[:, pl.ds(noff, ROWS_PER_BLK)],
                        dbuf.at[(j + NBUF - 1) & (NBUF - 1)], sem_blk)

                return scan_block(bstart + j, kb, fill)

            fill = lax.fori_loop(0, nloop, blk_body, jnp.int32(0))

            lax.cond(fill > 0, flush, lambda f: f, fill)

    return k


def _dot_kernel(batch):
    b_per_w = batch // NUM_WORKERS
    nchunk = b_per_w // 128
    mesh = plsc.VectorSubcoreMesh(core_axis_name="c", subcore_axis_name="s")

    @functools.partial(
        pl.kernel,
        mesh=mesh,
        out_type=jax.ShapeDtypeStruct((batch,), jnp.float32),
        scratch_types=[
            pltpu.VMEM((2, 128, 128), jnp.float32),
            pltpu.VMEM((2, 128, 128), jnp.float32),
            pltpu.VMEM((b_per_w,), jnp.float32),
            pltpu.SemaphoreType.DMA,
        ],
    )
    def k(uscr_hbm, iscr_hbm, out_hbm, urows_v, irows_v, out_v, sem):
        w = _worker_id()
        base = w * b_per_w

        def start(j):
            sl = pl.ds(base + j * 128, 128)
            cu = pltpu.async_copy(uscr_hbm.at[sl, :], urows_v.at[j % 2], sem)
            ci = pltpu.async_copy(iscr_hbm.at[sl, :], irows_v.at[j % 2], sem)
            return cu, ci

        lane = lax.iota(jnp.int32, 16)
        perms = [lane ^ sh for sh in (8, 4, 2, 1)]

        inflight = start(0)
        for j in range(nchunk):
            for c in inflight:
                c.wait()
            if j + 1 < nchunk:
                inflight = start(j + 1)
            cu = urows_v.at[j % 2]
            ci = irows_v.at[j % 2]

            def grp_body(g, _, cu=cu, ci=ci, j=j):
                acc = jnp.zeros((16,), jnp.float32)
                for r in range(16):
                    row = g * 16 + r
                    u0 = cu[row, pl.ds(0, 16)]
                    u1 = cu[row, pl.ds(16, 16)]
                    i0 = ci[row, pl.ds(0, 16)]
                    i1 = ci[row, pl.ds(16, 16)]
                    p = u0 * i0 + u1 * i1
                    for pm in perms:
                        p = p + p.at[pm].get(mode="promise_in_bounds")
                    acc = jnp.where(lane == r, p, acc)
                out_v[pl.ds(j * 128 + g * 16, 16)] = acc
                return 0

            lax.fori_loop(0, 8, grp_body, 0)

        pltpu.sync_copy(out_v, out_hbm.at[pl.ds(base, b_per_w)])

    return k


@functools.partial(jax.jit, static_argnames=("batch", "n_u", "n_i"))
def _mf(user, item, U, I, batch, n_u, n_i):
    UT = jnp.transpose(U)
    IT = jnp.transpose(I)
    uscr, iscr = _scan_extract_kernel(batch, n_u)(user, item, UT, IT)
    return _dot_kernel(batch)(uscr, iscr)


def kernel(user, item, U, I):
    return _mf(user, item, U, I, user.shape[0], U.shape[0], I.shape[0])
